# baseline (speedup 1.0000x reference)
import functools
import math

import jax
import jax.numpy as jnp
from jax import lax
from jax.experimental import pallas as pl
from jax.experimental.pallas import tpu as pltpu
from jax.experimental.pallas import tpu_sc as plsc

_L = 16
_CC = 128


def _k1_body(x_ref, wb_ref, wg_ref, bbt_ref, bgt_ref, feat_ref, sct_ref):
    f = jnp.dot(x_ref[...], wb_ref[...], preferred_element_type=jnp.float32)
    f = f + bbt_ref[0:1, :]
    feat_ref[...] = f
    s = lax.dot_general(wg_ref[...], f, (((0,), (1,)), ((), ())),
                        preferred_element_type=jnp.float32)
    sct_ref[...] = s + bgt_ref[:, 0:1]


def _backbone(x, Wb, Wg, bbt, bgt, B, D, E, br):
    grid = (B // br,)
    return pl.pallas_call(
        _k1_body,
        grid=grid,
        in_specs=[
            pl.BlockSpec((br, D), lambda i: (i, 0)),
            pl.BlockSpec((D, D), lambda i: (0, 0)),
            pl.BlockSpec((D, E), lambda i: (0, 0)),
            pl.BlockSpec((8, D), lambda i: (0, 0)),
            pl.BlockSpec((E, 128), lambda i: (0, 0)),
        ],
        out_specs=[
            pl.BlockSpec((br, D), lambda i: (i, 0)),
            pl.BlockSpec((E, br), lambda i: (0, i)),
        ],
        out_shape=[
            jax.ShapeDtypeStruct((B, D), jnp.float32),
            jax.ShapeDtypeStruct((E, B), jnp.float32),
        ],
    )(x, Wb, Wg, bbt, bgt)


def _make_select(B, E, M):
    mesh = plsc.VectorSubcoreMesh(core_axis_name="c", subcore_axis_name="s")
    nvec = B // _L
    tpc = B // 16

    def body(sct_hbm, idx_hbm, w_hbm,
             u_v, idx_v, mask_v, msl_v, wchunk_v, wtok_v, wsel_v,
             masks_sh, wtok_sh, sem):
        cid = lax.axis_index("c")
        sid = lax.axis_index("s")

        @pl.when(cid == 0)
        def _core0():
            @pl.when(sid < E)
            def _experts():
                e = sid
                pltpu.sync_copy(sct_hbm.at[e], u_v)

                def conv(i, c):
                    bits = u_v[pl.ds(i * _L, _L)]
                    neg = (bits >> jnp.uint32(31)) != jnp.uint32(0)
                    u = jnp.where(neg, ~bits, bits | jnp.uint32(0x80000000))
                    u_v[pl.ds(i * _L, _L)] = u
                    return c
                lax.fori_loop(0, nvec, conv, 0)

                def count_ge(t):
                    def cb(i, acc):
                        u = u_v[pl.ds(i * _L, _L)]
                        return acc + jnp.sum((u >= t).astype(jnp.int32))
                    return lax.fori_loop(0, nvec, cb, jnp.int32(0))

                def bs(bit, t):
                    shift = (jnp.int32(31) - bit).astype(jnp.uint32)
                    cand = t | (jnp.uint32(1) << shift)
                    cnt = count_ge(cand)
                    return jnp.where(cnt >= M, cand, t)
                thr = lax.fori_loop(0, 32, bs, jnp.uint32(0))

                def pass_a(i, off):
                    u = u_v[pl.ds(i * _L, _L)]
                    gt = u > thr
                    gti = gt.astype(jnp.int32)
                    pos = off + plsc.cumsum(gti) - gti
                    toks = lax.iota(jnp.int32, _L) + i * _L
                    plsc.store_scatter(idx_v, [pos], toks, mask=gt)
                    mask_v[pl.ds(i * _L, _L)] = gti
                    return off + jnp.sum(gti)
                off = lax.fori_loop(0, nvec, pass_a, jnp.int32(0))

                def pass_b(i, off):
                    u = u_v[pl.ds(i * _L, _L)]
                    eq = u == thr
                    eqi = eq.astype(jnp.int32)
                    pos = off + plsc.cumsum(eqi) - eqi
                    keep = eq & (pos < M)
                    toks = lax.iota(jnp.int32, _L) + i * _L
                    plsc.store_scatter(idx_v, [pos], toks, mask=keep)
                    old = mask_v[pl.ds(i * _L, _L)]
                    mask_v[pl.ds(i * _L, _L)] = old | keep.astype(jnp.int32)
                    return off + jnp.sum(eqi)
                lax.fori_loop(0, nvec, pass_b, off)

                pltpu.sync_copy(idx_v, idx_hbm.at[e])
                pltpu.sync_copy(mask_v, masks_sh.at[e])

            plsc.subcore_barrier()

            for e_ in range(E):
                pltpu.sync_copy(masks_sh.at[e_, pl.ds(sid * tpc, tpc)],
                                msl_v.at[e_])

            def wb_(j, c):
                acc = jnp.zeros((_L,), jnp.int32)
                for e_ in range(E):
                    acc = acc + msl_v[e_, pl.ds(j * _L, _L)]
                m = jnp.maximum(acc, 1).astype(jnp.float32)
                wchunk_v[pl.ds(j * _L, _L)] = 1.0 / m
                return c
            lax.fori_loop(0, tpc // _L, wb_, 0)
            pltpu.sync_copy(wchunk_v, wtok_sh.at[pl.ds(sid * tpc, tpc)])

            plsc.subcore_barrier()

            @pl.when(sid < E)
            def _wsel():
                e = sid
                pltpu.sync_copy(wtok_sh, wtok_v)

                def gw(p, c):
                    iv = idx_v[pl.ds(p * _L, _L)]
                    wsel_v[pl.ds(p * _L, _L)] = plsc.load_gather(wtok_v, [iv])
                    return c
                lax.fori_loop(0, M // _L, gw, 0)
                pltpu.sync_copy(wsel_v, w_hbm.at[e])

    return pl.kernel(
        body,
        compiler_params=pltpu.CompilerParams(needs_layout_passes=False),
        out_type=[jax.ShapeDtypeStruct((E, M), jnp.int32),
                  jax.ShapeDtypeStruct((E, M), jnp.float32)],
        mesh=mesh,
        scratch_types=[
            pltpu.VMEM((B,), jnp.uint32),
            pltpu.VMEM((M,), jnp.int32),
            pltpu.VMEM((B,), jnp.int32),
            pltpu.VMEM((E, tpc), jnp.int32),
            pltpu.VMEM((tpc,), jnp.float32),
            pltpu.VMEM((B,), jnp.float32),
            pltpu.VMEM((M,), jnp.float32),
            pltpu.VMEM_SHARED((E, B), jnp.int32),
            pltpu.VMEM_SHARED((B,), jnp.float32),
            pltpu.SemaphoreType.DMA,
        ],
    )


def _make_gather(B, D):
    mesh = plsc.VectorSubcoreMesh(core_axis_name="c", subcore_axis_name="s")
    rpt = B // 32
    rch = 32

    def body(feat_hbm, idxf_hbm, out_hbm, idx_v, rows_v, sem):
        wid = lax.axis_index("s") * 2 + lax.axis_index("c")
        base = wid * rpt
        pltpu.sync_copy(idxf_hbm.at[pl.ds(base, rpt)], idx_v)

        def gb(j, c):
            pltpu.async_copy(feat_hbm.at[idx_v.at[pl.ds(j * rch, rch)]],
                             rows_v, sem).wait()
            pltpu.sync_copy(rows_v, out_hbm.at[pl.ds(base + j * rch, rch)])
            return c
        lax.fori_loop(0, rpt // rch, gb, 0)

    return pl.kernel(
        body,
        compiler_params=pltpu.CompilerParams(needs_layout_passes=False),
        out_type=[jax.ShapeDtypeStruct((B, D), jnp.float32)],
        mesh=mesh,
        scratch_types=[
            pltpu.VMEM((rpt,), jnp.int32),
            pltpu.VMEM((rch, D), jnp.float32),
            pltpu.SemaphoreType.DMA,
        ],
    )


def _make_mlp(E, M, D, H, O, bh):
    nhb = H // bh
    nch = O // _CC

    def body(g_ref, w1_ref, b1t_ref, w2_ref, b2t_ref, ww_ref, out_ref,
             acc_ref):
        hb = pl.program_id(1)
        h = jnp.dot(g_ref[0], w1_ref[0], preferred_element_type=jnp.float32)
        h = jnp.maximum(h + b1t_ref[0, 0:1, :], 0.0)
        y = jnp.dot(h, w2_ref[0], preferred_element_type=jnp.float32)

        @pl.when(hb == 0)
        def _():
            acc_ref[...] = y

        @pl.when(hb > 0)
        def _():
            acc_ref[...] = acc_ref[...] + y

        @pl.when(hb == nhb - 1)
        def _():
            f = (acc_ref[...] + b2t_ref[0, 0:1, :]) * ww_ref[0][:, 0:1]
            for c in range(nch):
                out_ref[c] = f[:, c * _CC:(c + 1) * _CC]

    return pl.pallas_call(
        body,
        grid=(E, nhb),
        in_specs=[
            pl.BlockSpec((1, M, D), lambda e, h: (e, 0, 0)),
            pl.BlockSpec((1, D, bh), lambda e, h: (e, 0, h)),
            pl.BlockSpec((1, 8, bh), lambda e, h: (e, 0, h)),
            pl.BlockSpec((1, bh, O), lambda e, h: (e, h, 0)),
            pl.BlockSpec((1, 8, O), lambda e, h: (e, 0, 0)),
            pl.BlockSpec((1, M, 128), lambda e, h: (e, 0, 0)),
        ],
        out_specs=pl.BlockSpec((nch, M, _CC), lambda e, h: (0, e, 0)),
        out_shape=jax.ShapeDtypeStruct((nch, E * M, _CC), jnp.float32),
        scratch_shapes=[pltpu.VMEM((M, O), jnp.float32)],
    )


def _make_combine(B, O):
    mesh = plsc.VectorSubcoreMesh(core_axis_name="c", subcore_axis_name="s")
    ppt = B // 16
    cpc = (O // _CC) // 2

    def body(ych_hbm, idx2d_hbm, out_hbm, rows_v, zbuf_v, tok_v, acc_sh, sem):
        cid = lax.axis_index("c")
        sid = lax.axis_index("s")
        pbase = sid * ppt

        def zb(r, c):
            for k in range(_CC // _L):
                zbuf_v[r, pl.ds(k * _L, _L)] = jnp.zeros((_L,), jnp.float32)
            return c
        lax.fori_loop(0, 128, zb, 0)
        pltpu.sync_copy(idx2d_hbm.at[pl.ds(sid * (ppt // 128), ppt // 128)],
                        tok_v)

        def chunk(cc, c):
            cabs = cid * cpc + cc
            for q in range(ppt // 128):
                pltpu.sync_copy(zbuf_v, acc_sh.at[pl.ds(pbase + q * 128, 128)])
            plsc.subcore_barrier()
            for t in range(ppt // 256):
                pltpu.sync_copy(ych_hbm.at[cabs, pl.ds(pbase + t * 256, 256)],
                                rows_v)
                for q in range(2):
                    pltpu.sync_copy(rows_v.at[pl.ds(q * 128, 128)],
                                    acc_sh.at[tok_v.at[t * 2 + q]], add=True)
            plsc.subcore_barrier()
            for t in range(ppt // 256):
                pltpu.sync_copy(acc_sh.at[pl.ds(pbase + t * 256, 256)], rows_v)
                pltpu.sync_copy(rows_v,
                                out_hbm.at[pl.ds(pbase + t * 256, 256),
                                           pl.ds(cabs * _CC, _CC)])
            plsc.subcore_barrier()
            return c
        lax.fori_loop(0, cpc, chunk, 0)

    return pl.kernel(
        body,
        compiler_params=pltpu.CompilerParams(needs_layout_passes=False),
        out_type=[jax.ShapeDtypeStruct((B, O), jnp.float32)],
        mesh=mesh,
        scratch_types=[
            pltpu.VMEM((256, _CC), jnp.float32),
            pltpu.VMEM((128, _CC), jnp.float32),
            pltpu.VMEM((ppt // 128, 128), jnp.int32),
            pltpu.VMEM_SHARED((B, _CC), jnp.float32),
            pltpu.SemaphoreType.DMA,
        ],
    )


def kernel(x, Wb, bb, Wg, bg, W1, b1, W2, b2):
    B, D = x.shape
    E = Wg.shape[1]
    H = W1.shape[2]
    O = W2.shape[2]
    M = max(1, int(math.ceil(B / float(E))))

    bbt = jnp.broadcast_to(bb[None, :], (8, D))
    bgt = jnp.broadcast_to(bg[:, None], (E, 128))
    b1t = jnp.broadcast_to(b1[:, None, :], (E, 8, H))
    b2t = jnp.broadcast_to(b2[:, None, :], (E, 8, O))

    features, scoresT = _backbone(x, Wb, Wg, bbt, bgt, B, D, E,
                                  br=min(512, B))
    sct_bits = lax.bitcast_convert_type(scoresT, jnp.uint32)
    idx, w = _make_select(B, E, M)(sct_bits)
    idxf = idx.reshape(B)
    gathered = _make_gather(B, D)(features, idxf)[0]
    ww = jnp.broadcast_to(w.reshape(E, M, 1), (E, M, 128))
    ych = _make_mlp(E, M, D, H, O, bh=min(256, H))(
        gathered.reshape(E, M, D), W1, b1t, W2, b2t, ww)
    out = _make_combine(B, O)(ych, idxf.reshape(B // 128, 128))[0]
    return out


# speedup vs baseline: 1.0877x; 1.0877x over previous
import functools
import math

import jax
import jax.numpy as jnp
from jax import lax
from jax.experimental import pallas as pl
from jax.experimental.pallas import tpu as pltpu
from jax.experimental.pallas import tpu_sc as plsc

_L = 16
_CC = 128


def _k1_body(x_ref, wb_ref, wg_ref, bbt_ref, bgt_ref, feat_ref, sct_ref):
    f = jnp.dot(x_ref[...], wb_ref[...], preferred_element_type=jnp.float32)
    f = f + bbt_ref[0:1, :]
    feat_ref[...] = f
    s = lax.dot_general(wg_ref[...], f, (((0,), (1,)), ((), ())),
                        preferred_element_type=jnp.float32)
    sct_ref[...] = s + bgt_ref[:, 0:1]


def _backbone(x, Wb, Wg, bbt, bgt, B, D, E, br):
    grid = (B // br,)
    return pl.pallas_call(
        _k1_body,
        grid=grid,
        in_specs=[
            pl.BlockSpec((br, D), lambda i: (i, 0)),
            pl.BlockSpec((D, D), lambda i: (0, 0)),
            pl.BlockSpec((D, E), lambda i: (0, 0)),
            pl.BlockSpec((8, D), lambda i: (0, 0)),
            pl.BlockSpec((E, 128), lambda i: (0, 0)),
        ],
        out_specs=[
            pl.BlockSpec((br, D), lambda i: (i, 0)),
            pl.BlockSpec((E, br), lambda i: (0, i)),
        ],
        out_shape=[
            jax.ShapeDtypeStruct((B, D), jnp.float32),
            jax.ShapeDtypeStruct((E, B), jnp.float32),
        ],
    )(x, Wb, Wg, bbt, bgt)


def _make_select(B, E, M):
    mesh = plsc.VectorSubcoreMesh(core_axis_name="c", subcore_axis_name="s")
    nvec = B // _L
    tpc = B // 16

    def body(sct_hbm, idx_hbm, w_hbm,
             u_v, idx_v, mask_v, msl_v, wchunk_v, wtok_v, wsel_v,
             masks_sh, wtok_sh, sem):
        cid = lax.axis_index("c")
        sid = lax.axis_index("s")

        @pl.when(cid == 0)
        def _core0():
            @pl.when(sid < E)
            def _experts():
                e = sid
                pltpu.sync_copy(sct_hbm.at[e], u_v)

                def conv(i, c):
                    bits = u_v[pl.ds(i * _L, _L)]
                    neg = (bits >> jnp.uint32(31)) != jnp.uint32(0)
                    u = jnp.where(neg, ~bits, bits | jnp.uint32(0x80000000))
                    u_v[pl.ds(i * _L, _L)] = u
                    return c
                lax.fori_loop(0, nvec, conv, 0)

                def count_ge(t):
                    def cb(i, acc):
                        u = u_v[pl.ds(i * _L, _L)]
                        return acc + jnp.sum((u >= t).astype(jnp.int32))
                    return lax.fori_loop(0, nvec, cb, jnp.int32(0))

                def bs(bit, t):
                    shift = (jnp.int32(31) - bit).astype(jnp.uint32)
                    cand = t | (jnp.uint32(1) << shift)
                    cnt = count_ge(cand)
                    return jnp.where(cnt >= M, cand, t)
                thr = lax.fori_loop(0, 32, bs, jnp.uint32(0))

                def pass_a(i, off):
                    u = u_v[pl.ds(i * _L, _L)]
                    gt = u > thr
                    gti = gt.astype(jnp.int32)
                    pos = off + plsc.cumsum(gti) - gti
                    toks = lax.iota(jnp.int32, _L) + i * _L
                    plsc.store_scatter(idx_v, [pos], toks, mask=gt)
                    mask_v[pl.ds(i * _L, _L)] = gti
                    return off + jnp.sum(gti)
                off = lax.fori_loop(0, nvec, pass_a, jnp.int32(0))

                def pass_b(i, off):
                    u = u_v[pl.ds(i * _L, _L)]
                    eq = u == thr
                    eqi = eq.astype(jnp.int32)
                    pos = off + plsc.cumsum(eqi) - eqi
                    keep = eq & (pos < M)
                    toks = lax.iota(jnp.int32, _L) + i * _L
                    plsc.store_scatter(idx_v, [pos], toks, mask=keep)
                    old = mask_v[pl.ds(i * _L, _L)]
                    mask_v[pl.ds(i * _L, _L)] = old | keep.astype(jnp.int32)
                    return off + jnp.sum(eqi)
                lax.fori_loop(0, nvec, pass_b, off)

                pltpu.sync_copy(idx_v, idx_hbm.at[e])
                pltpu.sync_copy(mask_v, masks_sh.at[e])

            plsc.subcore_barrier()

            for e_ in range(E):
                pltpu.sync_copy(masks_sh.at[e_, pl.ds(sid * tpc, tpc)],
                                msl_v.at[e_])

            def wb_(j, c):
                acc = jnp.zeros((_L,), jnp.int32)
                for e_ in range(E):
                    acc = acc + msl_v[e_, pl.ds(j * _L, _L)]
                m = jnp.maximum(acc, 1).astype(jnp.float32)
                wchunk_v[pl.ds(j * _L, _L)] = 1.0 / m
                return c
            lax.fori_loop(0, tpc // _L, wb_, 0)
            pltpu.sync_copy(wchunk_v, wtok_sh.at[pl.ds(sid * tpc, tpc)])

            plsc.subcore_barrier()

            @pl.when(sid < E)
            def _wsel():
                e = sid
                pltpu.sync_copy(wtok_sh, wtok_v)

                def gw(p, c):
                    iv = idx_v[pl.ds(p * _L, _L)]
                    wsel_v[pl.ds(p * _L, _L)] = plsc.load_gather(wtok_v, [iv])
                    return c
                lax.fori_loop(0, M // _L, gw, 0)
                pltpu.sync_copy(wsel_v, w_hbm.at[e])

    return pl.kernel(
        body,
        compiler_params=pltpu.CompilerParams(needs_layout_passes=False),
        out_type=[jax.ShapeDtypeStruct((E, M), jnp.int32),
                  jax.ShapeDtypeStruct((E, M), jnp.float32)],
        mesh=mesh,
        scratch_types=[
            pltpu.VMEM((B,), jnp.uint32),
            pltpu.VMEM((M,), jnp.int32),
            pltpu.VMEM((B,), jnp.int32),
            pltpu.VMEM((E, tpc), jnp.int32),
            pltpu.VMEM((tpc,), jnp.float32),
            pltpu.VMEM((B,), jnp.float32),
            pltpu.VMEM((M,), jnp.float32),
            pltpu.VMEM_SHARED((E, B), jnp.int32),
            pltpu.VMEM_SHARED((B,), jnp.float32),
            pltpu.SemaphoreType.DMA,
        ],
    )


def _make_gather(B, D):
    mesh = plsc.VectorSubcoreMesh(core_axis_name="c", subcore_axis_name="s")
    rpt = B // 32
    rch = 32

    def body(feat_hbm, idxf_hbm, out_hbm, idx_v, rows_v, sem):
        wid = lax.axis_index("s") * 2 + lax.axis_index("c")
        base = wid * rpt
        pltpu.sync_copy(idxf_hbm.at[pl.ds(base, rpt)], idx_v)

        def gb(j, c):
            pltpu.async_copy(feat_hbm.at[idx_v.at[pl.ds(j * rch, rch)]],
                             rows_v, sem).wait()
            pltpu.sync_copy(rows_v, out_hbm.at[pl.ds(base + j * rch, rch)])
            return c
        lax.fori_loop(0, rpt // rch, gb, 0)

    return pl.kernel(
        body,
        compiler_params=pltpu.CompilerParams(needs_layout_passes=False),
        out_type=[jax.ShapeDtypeStruct((B, D), jnp.float32)],
        mesh=mesh,
        scratch_types=[
            pltpu.VMEM((rpt,), jnp.int32),
            pltpu.VMEM((rch, D), jnp.float32),
            pltpu.SemaphoreType.DMA,
        ],
    )


def _make_mlp(E, M, D, H, O, bh):
    nhb = H // bh
    nch = O // _CC

    def body(g_ref, w1_ref, b1t_ref, w2_ref, b2t_ref, ww_ref, out_ref,
             acc_ref):
        hb = pl.program_id(1)
        g16 = g_ref[0].astype(jnp.bfloat16)
        h = jnp.dot(g16, w1_ref[0], preferred_element_type=jnp.float32)
        h = jnp.maximum(h + b1t_ref[0, 0:1, :], 0.0)
        y = jnp.dot(h.astype(jnp.bfloat16), w2_ref[0],
                    preferred_element_type=jnp.float32)

        @pl.when(hb == 0)
        def _():
            acc_ref[...] = y

        @pl.when(hb > 0)
        def _():
            acc_ref[...] = acc_ref[...] + y

        @pl.when(hb == nhb - 1)
        def _():
            f = (acc_ref[...] + b2t_ref[0, 0:1, :]) * ww_ref[0][:, 0:1]
            for c in range(nch):
                out_ref[c] = f[:, c * _CC:(c + 1) * _CC]

    return pl.pallas_call(
        body,
        grid=(E, nhb),
        in_specs=[
            pl.BlockSpec((1, M, D), lambda e, h: (e, 0, 0)),
            pl.BlockSpec((1, D, bh), lambda e, h: (e, 0, h)),
            pl.BlockSpec((1, 8, bh), lambda e, h: (e, 0, h)),
            pl.BlockSpec((1, bh, O), lambda e, h: (e, h, 0)),
            pl.BlockSpec((1, 8, O), lambda e, h: (e, 0, 0)),
            pl.BlockSpec((1, M, 128), lambda e, h: (e, 0, 0)),
        ],
        out_specs=pl.BlockSpec((nch, M, _CC), lambda e, h: (0, e, 0)),
        out_shape=jax.ShapeDtypeStruct((nch, E * M, _CC), jnp.float32),
        scratch_shapes=[pltpu.VMEM((M, O), jnp.float32)],
    )


def _make_combine(B, O):
    mesh = plsc.VectorSubcoreMesh(core_axis_name="c", subcore_axis_name="s")
    ppt = B // 16
    cpc = (O // _CC) // 2

    def body(ych_hbm, idx2d_hbm, out_hbm, rows_v, zbuf_v, tok_v, acc_sh, sem):
        cid = lax.axis_index("c")
        sid = lax.axis_index("s")
        pbase = sid * ppt

        def zb(r, c):
            for k in range(_CC // _L):
                zbuf_v[r, pl.ds(k * _L, _L)] = jnp.zeros((_L,), jnp.float32)
            return c
        lax.fori_loop(0, 128, zb, 0)
        pltpu.sync_copy(idx2d_hbm.at[pl.ds(sid * (ppt // 128), ppt // 128)],
                        tok_v)

        def chunk(cc, c):
            cabs = cid * cpc + cc
            for q in range(ppt // 128):
                pltpu.sync_copy(zbuf_v, acc_sh.at[pl.ds(pbase + q * 128, 128)])
            plsc.subcore_barrier()
            for t in range(ppt // 256):
                pltpu.sync_copy(ych_hbm.at[cabs, pl.ds(pbase + t * 256, 256)],
                                rows_v)
                for q in range(2):
                    pltpu.sync_copy(rows_v.at[pl.ds(q * 128, 128)],
                                    acc_sh.at[tok_v.at[t * 2 + q]], add=True)
            plsc.subcore_barrier()
            for t in range(ppt // 256):
                pltpu.sync_copy(acc_sh.at[pl.ds(pbase + t * 256, 256)], rows_v)
                pltpu.sync_copy(rows_v,
                                out_hbm.at[pl.ds(pbase + t * 256, 256),
                                           pl.ds(cabs * _CC, _CC)])
            plsc.subcore_barrier()
            return c
        lax.fori_loop(0, cpc, chunk, 0)

    return pl.kernel(
        body,
        compiler_params=pltpu.CompilerParams(needs_layout_passes=False),
        out_type=[jax.ShapeDtypeStruct((B, O), jnp.float32)],
        mesh=mesh,
        scratch_types=[
            pltpu.VMEM((256, _CC), jnp.float32),
            pltpu.VMEM((128, _CC), jnp.float32),
            pltpu.VMEM((ppt // 128, 128), jnp.int32),
            pltpu.VMEM_SHARED((B, _CC), jnp.float32),
            pltpu.SemaphoreType.DMA,
        ],
    )


def kernel(x, Wb, bb, Wg, bg, W1, b1, W2, b2):
    B, D = x.shape
    E = Wg.shape[1]
    H = W1.shape[2]
    O = W2.shape[2]
    M = max(1, int(math.ceil(B / float(E))))

    bbt = jnp.broadcast_to(bb[None, :], (8, D))
    bgt = jnp.broadcast_to(bg[:, None], (E, 128))
    b1t = jnp.broadcast_to(b1[:, None, :], (E, 8, H))
    b2t = jnp.broadcast_to(b2[:, None, :], (E, 8, O))

    features, scoresT = _backbone(x, Wb, Wg, bbt, bgt, B, D, E,
                                  br=min(512, B))
    sct_bits = lax.bitcast_convert_type(scoresT, jnp.uint32)
    idx, w = _make_select(B, E, M)(sct_bits)
    idxf = idx.reshape(B)
    gathered = _make_gather(B, D)(features, idxf)[0]
    ww = jnp.broadcast_to(w.reshape(E, M, 1), (E, M, 128))
    ych = _make_mlp(E, M, D, H, O, bh=min(512, H))(
        gathered.reshape(E, M, D), W1.astype(jnp.bfloat16), b1t,
        W2.astype(jnp.bfloat16), b2t, ww)
    out = _make_combine(B, O)(ych, idxf.reshape(B // 128, 128))[0]
    return out
